# baseline (speedup 1.0000x reference)
import functools

import jax
import jax.numpy as jnp
from jax import lax
from jax.experimental import pallas as pl
from jax.experimental.pallas import tpu as pltpu
from jax.experimental.pallas import tpu_sc as plsc

NNODES = 10000
DIN = 128
DH = 64
NC = 2
NS = 16
NW = NC * NS
CHUNK = 128
RPT = 632
NP = NS * RPT
DEGW = 16

def _mesh():
    return plsc.VectorSubcoreMesh(
        core_axis_name="c", subcore_axis_name="s",
        num_cores=NC, num_subcores=NS)


def _zero_rows(ref, nrows, width):
    zero16 = jnp.zeros((16,), jnp.float32)

    def zrow(i, _):
        for j in range(width // 16):
            ref[i, pl.ds(j * 16, 16)] = zero16
        return 0

    lax.fori_loop(0, nrows, zrow, 0)


def _zero_acc_slice(zbuf, acc_sh, sid):
    nfull = RPT // CHUNK
    rem = RPT - nfull * CHUNK
    for j in range(nfull):
        pltpu.sync_copy(zbuf, acc_sh.at[pl.ds(sid * RPT + j * CHUNK, CHUNK)])
    if rem:
        pltpu.sync_copy(
            zbuf.at[pl.ds(0, rem)],
            acc_sh.at[pl.ds(sid * RPT + nfull * CHUNK, rem)])


def _deg_fast_body(nfw, tail, dst_hbm, out_hbm, *refs):
    if tail:
        (di0, di1, ones_v, dit, ones_t, buf_v, acc_sh, isem0, isem1) = refs
    else:
        (di0, di1, ones_v, buf_v, acc_sh, isem0, isem1) = refs
    ep = nfw * CHUNK + tail
    npairs = nfw // 2
    cid = lax.axis_index("c")
    sid = lax.axis_index("s")
    wid = cid * NS + sid
    base = wid * ep
    one16 = jnp.ones((16,), jnp.float32)
    zero16 = jnp.zeros((16,), jnp.float32)

    pltpu.async_copy(dst_hbm.at[pl.ds(base, CHUNK)], di0, isem0)
    pltpu.async_copy(dst_hbm.at[pl.ds(base + CHUNK, CHUNK)], di1, isem1)

    def fill(i, _):
        ones_v[i] = one16
        buf_v[i] = zero16
        return 0

    lax.fori_loop(0, CHUNK, fill, 0)
    if tail:
        def fillt(i, _):
            ones_t[i] = one16
            return 0

        lax.fori_loop(0, tail, fillt, 0)
    _zero_acc_slice(buf_v, acc_sh, sid)
    pltpu.make_async_copy(dst_hbm.at[pl.ds(base, CHUNK)], di0, isem0).wait()
    plsc.subcore_barrier()

    def pair(p, _):
        n0 = base + (2 * p + 2) * CHUNK
        n1 = n0 + CHUNK
        pltpu.sync_copy(ones_v, acc_sh.at[di0], add=True)
        pltpu.async_copy(dst_hbm.at[pl.ds(n0, CHUNK)], di0, isem0)
        pltpu.make_async_copy(
            dst_hbm.at[pl.ds(n0, CHUNK)], di1, isem1).wait()
        pltpu.sync_copy(ones_v, acc_sh.at[di1], add=True)
        pltpu.make_async_copy(
            dst_hbm.at[pl.ds(n0, CHUNK)], di0, isem0).wait()
        pltpu.async_copy(dst_hbm.at[pl.ds(n1, CHUNK)], di1, isem1)
        return 0

    lax.fori_loop(0, npairs - 1, pair, 0)
    pltpu.sync_copy(ones_v, acc_sh.at[di0], add=True)
    if tail:
        pltpu.async_copy(
            dst_hbm.at[pl.ds(base + nfw * CHUNK, tail)], dit, isem0)
    pltpu.make_async_copy(
        dst_hbm.at[pl.ds(base, CHUNK)], di1, isem1).wait()
    pltpu.sync_copy(ones_v, acc_sh.at[di1], add=True)
    if tail:
        pltpu.make_async_copy(
            dst_hbm.at[pl.ds(base, tail)], dit, isem0).wait()
        pltpu.sync_copy(ones_t, acc_sh.at[dit], add=True)
    plsc.subcore_barrier()
    pltpu.sync_copy(acc_sh.at[pl.ds(sid * RPT, RPT)],
                    out_hbm.at[cid, pl.ds(sid * RPT, RPT)])


def _msg_fast_body(nfw, tail, hs_hbm, src_hbm, dst_hbm, out_hbm, *refs):
    if tail:
        (si0, di0, si1, di1, rows0, rows1, sit, dit, rowst,
         rd_v, acc_sh, gsem0, gsem1, isem0, isem1, ssem0, ssem1) = refs
    else:
        (si0, di0, si1, di1, rows0, rows1,
         rd_v, acc_sh, gsem0, gsem1, isem0, isem1, ssem0, ssem1) = refs
    ep = nfw * CHUNK + tail
    npairs = nfw // 2
    cid = lax.axis_index("c")
    sid = lax.axis_index("s")
    wid = cid * NS + sid
    base = wid * ep

    pltpu.async_copy(src_hbm.at[pl.ds(base, CHUNK)], si0, isem0)
    pltpu.async_copy(dst_hbm.at[pl.ds(base, CHUNK)], di0, isem0)
    pltpu.async_copy(src_hbm.at[pl.ds(base + CHUNK, CHUNK)], si1, isem1)
    _zero_rows(rd_v, CHUNK, DH)
    pltpu.make_async_copy(src_hbm.at[pl.ds(base, CHUNK)], si0, isem0).wait()
    pltpu.make_async_copy(dst_hbm.at[pl.ds(base, CHUNK)], di0, isem0).wait()
    pltpu.async_copy(hs_hbm.at[si0], rows0, gsem0)
    _zero_acc_slice(rd_v, acc_sh, sid)
    plsc.subcore_barrier()
    pltpu.async_copy(rd_v, acc_sh.at[di0], ssem1, add=True)

    def pair(p, _):
        b0 = base + (2 * p + 1) * CHUNK
        n0 = b0 + CHUNK
        n1 = n0 + CHUNK
        pltpu.make_async_copy(
            src_hbm.at[pl.ds(b0, CHUNK)], si1, isem1).wait()
        pltpu.make_async_copy(rows1, acc_sh.at[di1], ssem1).wait()
        pltpu.async_copy(hs_hbm.at[si1], rows1, gsem1)
        pltpu.async_copy(dst_hbm.at[pl.ds(b0, CHUNK)], di1, isem1)
        pltpu.make_async_copy(hs_hbm.at[si0], rows0, gsem0).wait()
        pltpu.async_copy(rows0, acc_sh.at[di0], ssem0, add=True)
        pltpu.async_copy(src_hbm.at[pl.ds(n0, CHUNK)], si0, isem0)
        pltpu.make_async_copy(hs_hbm.at[si1], rows1, gsem1).wait()
        pltpu.make_async_copy(
            dst_hbm.at[pl.ds(b0, CHUNK)], di1, isem1).wait()
        pltpu.make_async_copy(rows0, acc_sh.at[di0], ssem0).wait()
        pltpu.async_copy(rows1, acc_sh.at[di1], ssem1, add=True)
        pltpu.async_copy(dst_hbm.at[pl.ds(n0, CHUNK)], di0, isem0)
        pltpu.make_async_copy(
            src_hbm.at[pl.ds(n0, CHUNK)], si0, isem0).wait()
        pltpu.make_async_copy(
            dst_hbm.at[pl.ds(n0, CHUNK)], di0, isem0).wait()
        pltpu.async_copy(hs_hbm.at[si0], rows0, gsem0)
        pltpu.async_copy(src_hbm.at[pl.ds(n1, CHUNK)], si1, isem1)
        return 0

    lax.fori_loop(0, npairs - 1, pair, 0)

    bL = base + (nfw - 1) * CHUNK
    pltpu.make_async_copy(src_hbm.at[pl.ds(bL, CHUNK)], si1, isem1).wait()
    pltpu.make_async_copy(rows1, acc_sh.at[di1], ssem1).wait()
    pltpu.async_copy(hs_hbm.at[si1], rows1, gsem1)
    pltpu.async_copy(dst_hbm.at[pl.ds(bL, CHUNK)], di1, isem1)
    pltpu.make_async_copy(hs_hbm.at[si0], rows0, gsem0).wait()
    pltpu.async_copy(rows0, acc_sh.at[di0], ssem0, add=True)
    if tail:
        pltpu.async_copy(
            src_hbm.at[pl.ds(base + nfw * CHUNK, tail)], sit, isem0)
        pltpu.async_copy(
            dst_hbm.at[pl.ds(base + nfw * CHUNK, tail)], dit, isem0)
    pltpu.make_async_copy(hs_hbm.at[si1], rows1, gsem1).wait()
    pltpu.make_async_copy(dst_hbm.at[pl.ds(bL, CHUNK)], di1, isem1).wait()
    pltpu.make_async_copy(rows0, acc_sh.at[di0], ssem0).wait()
    pltpu.async_copy(rows1, acc_sh.at[di1], ssem1, add=True)
    if tail:
        pltpu.make_async_copy(
            src_hbm.at[pl.ds(base, tail)], sit, isem0).wait()
        pltpu.make_async_copy(
            dst_hbm.at[pl.ds(base, tail)], dit, isem0).wait()
        pltpu.async_copy(hs_hbm.at[sit], rowst, gsem0).wait()
        pltpu.sync_copy(rowst, acc_sh.at[dit], add=True)
    pltpu.make_async_copy(rows1, acc_sh.at[di1], ssem1).wait()
    plsc.subcore_barrier()
    pltpu.sync_copy(acc_sh.at[pl.ds(sid * RPT, RPT)],
                    out_hbm.at[cid, pl.ds(sid * RPT, RPT)])


def _make_deg_fast(nfw, tail):
    scratch = [
        pltpu.VMEM((CHUNK,), jnp.int32),
        pltpu.VMEM((CHUNK,), jnp.int32),
        pltpu.VMEM((CHUNK, DEGW), jnp.float32),
    ]
    if tail:
        scratch += [
            pltpu.VMEM((tail,), jnp.int32),
            pltpu.VMEM((tail, DEGW), jnp.float32),
        ]
    scratch += [
        pltpu.VMEM((CHUNK, DEGW), jnp.float32),
        pltpu.VMEM_SHARED((NP, DEGW), jnp.float32),
        pltpu.SemaphoreType.DMA,
        pltpu.SemaphoreType.DMA,
    ]
    return pl.kernel(
        functools.partial(_deg_fast_body, nfw, tail),
        out_type=jax.ShapeDtypeStruct((NC, NP, DEGW), jnp.float32),
        mesh=_mesh(),
        scratch_types=scratch,
        compiler_params=pltpu.CompilerParams(use_tc_tiling_on_sc=False),
        name="gcn_degree_sc",
    )


def _make_msg_fast(nfw, tail):
    scratch = [
        pltpu.VMEM((CHUNK,), jnp.int32),
        pltpu.VMEM((CHUNK,), jnp.int32),
        pltpu.VMEM((CHUNK,), jnp.int32),
        pltpu.VMEM((CHUNK,), jnp.int32),
        pltpu.VMEM((CHUNK, DH), jnp.float32),
        pltpu.VMEM((CHUNK, DH), jnp.float32),
    ]
    if tail:
        scratch += [
            pltpu.VMEM((tail,), jnp.int32),
            pltpu.VMEM((tail,), jnp.int32),
            pltpu.VMEM((tail, DH), jnp.float32),
        ]
    scratch += [
        pltpu.VMEM((CHUNK, DH), jnp.float32),
        pltpu.VMEM_SHARED((NP, DH), jnp.float32),
        pltpu.SemaphoreType.DMA,
        pltpu.SemaphoreType.DMA,
        pltpu.SemaphoreType.DMA,
        pltpu.SemaphoreType.DMA,
        pltpu.SemaphoreType.DMA,
        pltpu.SemaphoreType.DMA,
    ]
    return pl.kernel(
        functools.partial(_msg_fast_body, nfw, tail),
        out_type=jax.ShapeDtypeStruct((NC, NP, DH), jnp.float32),
        mesh=_mesh(),
        scratch_types=scratch,
        compiler_params=pltpu.CompilerParams(use_tc_tiling_on_sc=False),
        name="gcn_message_sc",
    )


def _deg_body(nwin, dstw_hbm, out_hbm, di_v, ones_v, buf_v, acc_sh):
    cid = lax.axis_index("c")
    sid = lax.axis_index("s")
    wid = cid * NS + sid
    one16 = jnp.ones((16,), jnp.float32)

    def fill(i, _):
        ones_v[i] = one16
        return 0

    lax.fori_loop(0, CHUNK, fill, 0)
    _zero_rows(buf_v, RPT, DEGW)
    pltpu.sync_copy(buf_v, acc_sh.at[pl.ds(sid * RPT, RPT)])
    plsc.subcore_barrier()

    def step(k, _):
        pltpu.sync_copy(dstw_hbm.at[wid, k], di_v)
        pltpu.sync_copy(ones_v, acc_sh.at[di_v], add=True)
        return 0

    lax.fori_loop(0, nwin, step, 0)
    plsc.subcore_barrier()
    pltpu.sync_copy(acc_sh.at[pl.ds(sid * RPT, RPT)], buf_v)
    pltpu.sync_copy(buf_v, out_hbm.at[cid, pl.ds(sid * RPT, RPT)])


def _msg_body(nwin, hs_hbm, srcw_hbm, dstw_hbm, out_hbm,
              si_v, di_v, rows_v, rd_v, acc_sh, sem):
    cid = lax.axis_index("c")
    sid = lax.axis_index("s")
    wid = cid * NS + sid

    _zero_rows(rd_v, RPT, DH)
    pltpu.sync_copy(rd_v, acc_sh.at[pl.ds(sid * RPT, RPT)])
    plsc.subcore_barrier()

    def step(k, _):
        pltpu.sync_copy(srcw_hbm.at[wid, k], si_v)
        pltpu.sync_copy(dstw_hbm.at[wid, k], di_v)
        pltpu.async_copy(hs_hbm.at[si_v], rows_v, sem).wait()
        pltpu.sync_copy(rows_v, acc_sh.at[di_v], add=True)
        return 0

    lax.fori_loop(0, nwin, step, 0)
    plsc.subcore_barrier()
    pltpu.sync_copy(acc_sh.at[pl.ds(sid * RPT, RPT)], rd_v)
    pltpu.sync_copy(rd_v, out_hbm.at[cid, pl.ds(sid * RPT, RPT)])


def _make_deg_kernel(nwin):
    return pl.kernel(
        functools.partial(_deg_body, nwin),
        out_type=jax.ShapeDtypeStruct((NC, NP, DEGW), jnp.float32),
        mesh=_mesh(),
        scratch_types=[
            pltpu.VMEM((CHUNK,), jnp.int32),
            pltpu.VMEM((CHUNK, DEGW), jnp.float32),
            pltpu.VMEM((RPT, DEGW), jnp.float32),
            pltpu.VMEM_SHARED((NP, DEGW), jnp.float32),
        ],
        compiler_params=pltpu.CompilerParams(use_tc_tiling_on_sc=False),
        name="gcn_degree_sc",
    )


def _make_msg_kernel(nwin):
    return pl.kernel(
        functools.partial(_msg_body, nwin),
        out_type=jax.ShapeDtypeStruct((NC, NP, DH), jnp.float32),
        mesh=_mesh(),
        scratch_types=[
            pltpu.VMEM((CHUNK,), jnp.int32),
            pltpu.VMEM((CHUNK,), jnp.int32),
            pltpu.VMEM((CHUNK, DH), jnp.float32),
            pltpu.VMEM((RPT, DH), jnp.float32),
            pltpu.VMEM_SHARED((NP, DH), jnp.float32),
            pltpu.SemaphoreType.DMA,
        ],
        compiler_params=pltpu.CompilerParams(use_tc_tiling_on_sc=False),
        name="gcn_message_sc",
    )


def _dis_from_degp(degp):
    return lax.rsqrt(degp[0, :, 0] + degp[1, :, 0] + 1.0)


def _tca1_body(x_ref, w1_ref, h_ref):
    h_ref[...] = jnp.dot(
        x_ref[...], w1_ref[...], preferred_element_type=jnp.float32)


def _tca2_body(h_ref, degp_ref, hs_ref):
    dis = _dis_from_degp(degp_ref[...])
    hs_ref[...] = h_ref[...] * dis[:, None]


def _tcb_body(msgp_ref, hs_ref, degp_ref, b_ref, w2_ref, out_ref):
    dis = _dis_from_degp(degp_ref[...])
    tot = msgp_ref[0] + msgp_ref[1] + hs_ref[...]
    h = jnp.maximum(tot * dis[:, None] + b_ref[...], 0.0)
    out_ref[...] = jnp.dot(
        h, w2_ref[...], preferred_element_type=jnp.float32) * dis[:, None]


def _tcc_body(msgp_ref, hs_ref, degp_ref, b_ref, wfct_ref, bfc_ref, out_ref):
    dis = _dis_from_degp(degp_ref[...])
    tot = msgp_ref[0] + msgp_ref[1] + hs_ref[...]
    h = jnp.maximum(tot * dis[:, None] + b_ref[...], 0.0)
    rows = lax.broadcasted_iota(jnp.int32, (NP, 1), 0)
    h = jnp.where(rows < NNODES, h, 0.0)
    g = jnp.sum(h, axis=0, keepdims=True) * (1.0 / NNODES)
    z = jnp.sum(g * wfct_ref[...], axis=1, keepdims=True) + bfc_ref[...]
    out_ref[...] = jax.nn.sigmoid(z)


def kernel(x, edge_index, W1, b1, W2, b2, Wfc, bfc):
    nedges = edge_index.shape[1]
    src = edge_index[0]
    dst = edge_index[1]
    x_pad = jnp.concatenate(
        [x, jnp.zeros((NP - NNODES, DIN), jnp.float32)], axis=0)

    ep = nedges // NW
    nfw = ep // CHUNK
    tail = ep - nfw * CHUNK
    fast = (nedges == ep * NW and ep % 8 == 0 and tail % 8 == 0
            and nfw >= 2 and nfw % 2 == 0)

    if fast:
        degp = _make_deg_fast(nfw, tail)(dst)
        msg_kernel = _make_msg_fast(nfw, tail)
        msg1_args = msg2_args = (src, dst)
    else:
        nwin = -(-nedges // (NW * CHUNK))
        epad = NW * nwin * CHUNK - nedges
        srcw = jnp.concatenate(
            [src, jnp.full((epad,), NNODES, jnp.int32)]
        ).reshape(NW, nwin, CHUNK)
        dstw = jnp.concatenate(
            [dst, jnp.full((epad,), NNODES + 8, jnp.int32)]
        ).reshape(NW, nwin, CHUNK)
        degp = _make_deg_kernel(nwin)(dstw)
        msg_kernel = _make_msg_kernel(nwin)
        msg1_args = msg2_args = (srcw, dstw)

    h1 = pl.pallas_call(
        _tca1_body,
        out_shape=jax.ShapeDtypeStruct((NP, DH), jnp.float32),
    )(x_pad, W1)

    hs1 = pl.pallas_call(
        _tca2_body,
        out_shape=jax.ShapeDtypeStruct((NP, DH), jnp.float32),
    )(h1, degp)

    msgp1 = msg_kernel(hs1, *msg1_args)

    hs2 = pl.pallas_call(
        _tcb_body,
        out_shape=jax.ShapeDtypeStruct((NP, DH), jnp.float32),
    )(msgp1, hs1, degp, b1.reshape(1, DH), W2)

    msgp2 = msg_kernel(hs2, *msg2_args)

    out = pl.pallas_call(
        _tcc_body,
        out_shape=jax.ShapeDtypeStruct((1, 1), jnp.float32),
    )(msgp2, hs2, degp, b2.reshape(1, DH), Wfc.T.reshape(1, DH),
      bfc.reshape(1, 1))

    return out.reshape(1)


# speedup vs baseline: 1.0132x; 1.0132x over previous
import functools

import jax
import jax.numpy as jnp
from jax import lax
from jax.experimental import pallas as pl
from jax.experimental.pallas import tpu as pltpu
from jax.experimental.pallas import tpu_sc as plsc

NNODES = 10000
DIN = 128
DH = 64
NC = 2
NS = 16
NW = NC * NS
CHUNK = 128
RPT = 632
NP = NS * RPT
DEGW = 16

def _mesh():
    return plsc.VectorSubcoreMesh(
        core_axis_name="c", subcore_axis_name="s",
        num_cores=NC, num_subcores=NS)


def _zero_rows(ref, nrows, width):
    zero16 = jnp.zeros((16,), jnp.float32)

    def zrow(i, _):
        for j in range(width // 16):
            ref[i, pl.ds(j * 16, 16)] = zero16
        return 0

    lax.fori_loop(0, nrows, zrow, 0)


def _zero_acc_slice(zbuf, acc_sh, sid):
    nfull = RPT // CHUNK
    rem = RPT - nfull * CHUNK
    for j in range(nfull):
        pltpu.sync_copy(zbuf, acc_sh.at[pl.ds(sid * RPT + j * CHUNK, CHUNK)])
    if rem:
        pltpu.sync_copy(
            zbuf.at[pl.ds(0, rem)],
            acc_sh.at[pl.ds(sid * RPT + nfull * CHUNK, rem)])


def _deg_fast_body(nfw, tail, dst_hbm, out_hbm, *refs):
    if tail:
        (di0, di1, ones_v, dit, ones_t, buf_v, acc_sh, isem0, isem1) = refs
    else:
        (di0, di1, ones_v, buf_v, acc_sh, isem0, isem1) = refs
    ep = nfw * CHUNK + tail
    npairs = nfw // 2
    cid = lax.axis_index("c")
    sid = lax.axis_index("s")
    wid = cid * NS + sid
    base = wid * ep
    one16 = jnp.ones((16,), jnp.float32)
    zero16 = jnp.zeros((16,), jnp.float32)

    pltpu.async_copy(dst_hbm.at[pl.ds(base, CHUNK)], di0, isem0)
    pltpu.async_copy(dst_hbm.at[pl.ds(base + CHUNK, CHUNK)], di1, isem1)

    def fill(i, _):
        ones_v[i] = one16
        buf_v[i] = zero16
        return 0

    lax.fori_loop(0, CHUNK, fill, 0)
    if tail:
        def fillt(i, _):
            ones_t[i] = one16
            return 0

        lax.fori_loop(0, tail, fillt, 0)
    _zero_acc_slice(buf_v, acc_sh, sid)
    pltpu.make_async_copy(dst_hbm.at[pl.ds(base, CHUNK)], di0, isem0).wait()
    plsc.subcore_barrier()

    def pair(p, _):
        n0 = base + (2 * p + 2) * CHUNK
        n1 = n0 + CHUNK
        pltpu.sync_copy(ones_v, acc_sh.at[di0], add=True)
        pltpu.async_copy(dst_hbm.at[pl.ds(n0, CHUNK)], di0, isem0)
        pltpu.make_async_copy(
            dst_hbm.at[pl.ds(n0, CHUNK)], di1, isem1).wait()
        pltpu.sync_copy(ones_v, acc_sh.at[di1], add=True)
        pltpu.make_async_copy(
            dst_hbm.at[pl.ds(n0, CHUNK)], di0, isem0).wait()
        pltpu.async_copy(dst_hbm.at[pl.ds(n1, CHUNK)], di1, isem1)
        return 0

    lax.fori_loop(0, npairs - 1, pair, 0)
    pltpu.sync_copy(ones_v, acc_sh.at[di0], add=True)
    if tail:
        pltpu.async_copy(
            dst_hbm.at[pl.ds(base + nfw * CHUNK, tail)], dit, isem0)
    pltpu.make_async_copy(
        dst_hbm.at[pl.ds(base, CHUNK)], di1, isem1).wait()
    pltpu.sync_copy(ones_v, acc_sh.at[di1], add=True)
    if tail:
        pltpu.make_async_copy(
            dst_hbm.at[pl.ds(base, tail)], dit, isem0).wait()
        pltpu.sync_copy(ones_t, acc_sh.at[dit], add=True)
    plsc.subcore_barrier()
    pltpu.sync_copy(acc_sh.at[pl.ds(sid * RPT, RPT)],
                    out_hbm.at[cid, pl.ds(sid * RPT, RPT)])


def _msg_fast_body(nfw, tail, hs_hbm, src_hbm, dst_hbm, out_hbm, *refs):
    if tail:
        (si0, di0, si1, di1, rows0, rows1, sit, dit, rowst,
         rd_v, acc_sh, gsem0, gsem1, isem0, isem1) = refs
    else:
        (si0, di0, si1, di1, rows0, rows1,
         rd_v, acc_sh, gsem0, gsem1, isem0, isem1) = refs
    ep = nfw * CHUNK + tail
    npairs = nfw // 2
    cid = lax.axis_index("c")
    sid = lax.axis_index("s")
    wid = cid * NS + sid
    base = wid * ep

    pltpu.async_copy(src_hbm.at[pl.ds(base, CHUNK)], si0, isem0)
    pltpu.async_copy(dst_hbm.at[pl.ds(base, CHUNK)], di0, isem0)
    pltpu.async_copy(src_hbm.at[pl.ds(base + CHUNK, CHUNK)], si1, isem1)
    pltpu.async_copy(dst_hbm.at[pl.ds(base + CHUNK, CHUNK)], di1, isem1)
    _zero_rows(rd_v, CHUNK, DH)
    pltpu.make_async_copy(src_hbm.at[pl.ds(base, CHUNK)], si0, isem0).wait()
    pltpu.make_async_copy(dst_hbm.at[pl.ds(base, CHUNK)], di0, isem0).wait()
    pltpu.async_copy(hs_hbm.at[si0], rows0, gsem0)
    _zero_acc_slice(rd_v, acc_sh, sid)
    plsc.subcore_barrier()

    def pair(p, _):
        b0 = base + (2 * p + 1) * CHUNK
        n0 = b0 + CHUNK
        n1 = n0 + CHUNK
        pltpu.make_async_copy(
            src_hbm.at[pl.ds(b0, CHUNK)], si1, isem1).wait()
        pltpu.make_async_copy(
            dst_hbm.at[pl.ds(b0, CHUNK)], di1, isem1).wait()
        pltpu.async_copy(hs_hbm.at[si1], rows1, gsem1)
        pltpu.make_async_copy(hs_hbm.at[si0], rows0, gsem0).wait()
        pltpu.sync_copy(rows0, acc_sh.at[di0], add=True)
        pltpu.async_copy(src_hbm.at[pl.ds(n0, CHUNK)], si0, isem0)
        pltpu.async_copy(dst_hbm.at[pl.ds(n0, CHUNK)], di0, isem0)
        pltpu.make_async_copy(hs_hbm.at[si1], rows1, gsem1).wait()
        pltpu.sync_copy(rows1, acc_sh.at[di1], add=True)
        pltpu.make_async_copy(
            src_hbm.at[pl.ds(n0, CHUNK)], si0, isem0).wait()
        pltpu.make_async_copy(
            dst_hbm.at[pl.ds(n0, CHUNK)], di0, isem0).wait()
        pltpu.async_copy(hs_hbm.at[si0], rows0, gsem0)
        pltpu.async_copy(src_hbm.at[pl.ds(n1, CHUNK)], si1, isem1)
        pltpu.async_copy(dst_hbm.at[pl.ds(n1, CHUNK)], di1, isem1)
        return 0

    lax.fori_loop(0, npairs - 1, pair, 0)

    bL = base + (nfw - 1) * CHUNK
    pltpu.make_async_copy(src_hbm.at[pl.ds(bL, CHUNK)], si1, isem1).wait()
    pltpu.make_async_copy(dst_hbm.at[pl.ds(bL, CHUNK)], di1, isem1).wait()
    pltpu.async_copy(hs_hbm.at[si1], rows1, gsem1)
    pltpu.make_async_copy(hs_hbm.at[si0], rows0, gsem0).wait()
    pltpu.sync_copy(rows0, acc_sh.at[di0], add=True)
    if tail:
        pltpu.async_copy(
            src_hbm.at[pl.ds(base + nfw * CHUNK, tail)], sit, isem0)
        pltpu.async_copy(
            dst_hbm.at[pl.ds(base + nfw * CHUNK, tail)], dit, isem0)
    pltpu.make_async_copy(hs_hbm.at[si1], rows1, gsem1).wait()
    pltpu.sync_copy(rows1, acc_sh.at[di1], add=True)
    if tail:
        pltpu.make_async_copy(
            src_hbm.at[pl.ds(base, tail)], sit, isem0).wait()
        pltpu.make_async_copy(
            dst_hbm.at[pl.ds(base, tail)], dit, isem0).wait()
        pltpu.async_copy(hs_hbm.at[sit], rowst, gsem0).wait()
        pltpu.sync_copy(rowst, acc_sh.at[dit], add=True)
    plsc.subcore_barrier()
    pltpu.sync_copy(acc_sh.at[pl.ds(sid * RPT, RPT)],
                    out_hbm.at[cid, pl.ds(sid * RPT, RPT)])


def _make_deg_fast(nfw, tail):
    scratch = [
        pltpu.VMEM((CHUNK,), jnp.int32),
        pltpu.VMEM((CHUNK,), jnp.int32),
        pltpu.VMEM((CHUNK, DEGW), jnp.float32),
    ]
    if tail:
        scratch += [
            pltpu.VMEM((tail,), jnp.int32),
            pltpu.VMEM((tail, DEGW), jnp.float32),
        ]
    scratch += [
        pltpu.VMEM((CHUNK, DEGW), jnp.float32),
        pltpu.VMEM_SHARED((NP, DEGW), jnp.float32),
        pltpu.SemaphoreType.DMA,
        pltpu.SemaphoreType.DMA,
    ]
    return pl.kernel(
        functools.partial(_deg_fast_body, nfw, tail),
        out_type=jax.ShapeDtypeStruct((NC, NP, DEGW), jnp.float32),
        mesh=_mesh(),
        scratch_types=scratch,
        compiler_params=pltpu.CompilerParams(use_tc_tiling_on_sc=False),
        name="gcn_degree_sc",
    )


def _make_msg_fast(nfw, tail):
    scratch = [
        pltpu.VMEM((CHUNK,), jnp.int32),
        pltpu.VMEM((CHUNK,), jnp.int32),
        pltpu.VMEM((CHUNK,), jnp.int32),
        pltpu.VMEM((CHUNK,), jnp.int32),
        pltpu.VMEM((CHUNK, DH), jnp.float32),
        pltpu.VMEM((CHUNK, DH), jnp.float32),
    ]
    if tail:
        scratch += [
            pltpu.VMEM((tail,), jnp.int32),
            pltpu.VMEM((tail,), jnp.int32),
            pltpu.VMEM((tail, DH), jnp.float32),
        ]
    scratch += [
        pltpu.VMEM((CHUNK, DH), jnp.float32),
        pltpu.VMEM_SHARED((NP, DH), jnp.float32),
        pltpu.SemaphoreType.DMA,
        pltpu.SemaphoreType.DMA,
        pltpu.SemaphoreType.DMA,
        pltpu.SemaphoreType.DMA,
    ]
    return pl.kernel(
        functools.partial(_msg_fast_body, nfw, tail),
        out_type=jax.ShapeDtypeStruct((NC, NP, DH), jnp.float32),
        mesh=_mesh(),
        scratch_types=scratch,
        compiler_params=pltpu.CompilerParams(use_tc_tiling_on_sc=False),
        name="gcn_message_sc",
    )


def _deg_body(nwin, dstw_hbm, out_hbm, di_v, ones_v, buf_v, acc_sh):
    cid = lax.axis_index("c")
    sid = lax.axis_index("s")
    wid = cid * NS + sid
    one16 = jnp.ones((16,), jnp.float32)

    def fill(i, _):
        ones_v[i] = one16
        return 0

    lax.fori_loop(0, CHUNK, fill, 0)
    _zero_rows(buf_v, RPT, DEGW)
    pltpu.sync_copy(buf_v, acc_sh.at[pl.ds(sid * RPT, RPT)])
    plsc.subcore_barrier()

    def step(k, _):
        pltpu.sync_copy(dstw_hbm.at[wid, k], di_v)
        pltpu.sync_copy(ones_v, acc_sh.at[di_v], add=True)
        return 0

    lax.fori_loop(0, nwin, step, 0)
    plsc.subcore_barrier()
    pltpu.sync_copy(acc_sh.at[pl.ds(sid * RPT, RPT)], buf_v)
    pltpu.sync_copy(buf_v, out_hbm.at[cid, pl.ds(sid * RPT, RPT)])


def _msg_body(nwin, hs_hbm, srcw_hbm, dstw_hbm, out_hbm,
              si_v, di_v, rows_v, rd_v, acc_sh, sem):
    cid = lax.axis_index("c")
    sid = lax.axis_index("s")
    wid = cid * NS + sid

    _zero_rows(rd_v, RPT, DH)
    pltpu.sync_copy(rd_v, acc_sh.at[pl.ds(sid * RPT, RPT)])
    plsc.subcore_barrier()

    def step(k, _):
        pltpu.sync_copy(srcw_hbm.at[wid, k], si_v)
        pltpu.sync_copy(dstw_hbm.at[wid, k], di_v)
        pltpu.async_copy(hs_hbm.at[si_v], rows_v, sem).wait()
        pltpu.sync_copy(rows_v, acc_sh.at[di_v], add=True)
        return 0

    lax.fori_loop(0, nwin, step, 0)
    plsc.subcore_barrier()
    pltpu.sync_copy(acc_sh.at[pl.ds(sid * RPT, RPT)], rd_v)
    pltpu.sync_copy(rd_v, out_hbm.at[cid, pl.ds(sid * RPT, RPT)])


def _make_deg_kernel(nwin):
    return pl.kernel(
        functools.partial(_deg_body, nwin),
        out_type=jax.ShapeDtypeStruct((NC, NP, DEGW), jnp.float32),
        mesh=_mesh(),
        scratch_types=[
            pltpu.VMEM((CHUNK,), jnp.int32),
            pltpu.VMEM((CHUNK, DEGW), jnp.float32),
            pltpu.VMEM((RPT, DEGW), jnp.float32),
            pltpu.VMEM_SHARED((NP, DEGW), jnp.float32),
        ],
        compiler_params=pltpu.CompilerParams(use_tc_tiling_on_sc=False),
        name="gcn_degree_sc",
    )


def _make_msg_kernel(nwin):
    return pl.kernel(
        functools.partial(_msg_body, nwin),
        out_type=jax.ShapeDtypeStruct((NC, NP, DH), jnp.float32),
        mesh=_mesh(),
        scratch_types=[
            pltpu.VMEM((CHUNK,), jnp.int32),
            pltpu.VMEM((CHUNK,), jnp.int32),
            pltpu.VMEM((CHUNK, DH), jnp.float32),
            pltpu.VMEM((RPT, DH), jnp.float32),
            pltpu.VMEM_SHARED((NP, DH), jnp.float32),
            pltpu.SemaphoreType.DMA,
        ],
        compiler_params=pltpu.CompilerParams(use_tc_tiling_on_sc=False),
        name="gcn_message_sc",
    )


def _dis_from_degp(degp):
    return lax.rsqrt(degp[0, :, 0] + degp[1, :, 0] + 1.0)


def _tca1_body(x_ref, w1_ref, h_ref):
    h_ref[...] = jnp.dot(
        x_ref[...], w1_ref[...], preferred_element_type=jnp.float32)


def _tca2_body(h_ref, degp_ref, hs_ref):
    dis = _dis_from_degp(degp_ref[...])
    hs_ref[...] = h_ref[...] * dis[:, None]


def _tcb_body(msgp_ref, hs_ref, degp_ref, b_ref, w2_ref, out_ref):
    dis = _dis_from_degp(degp_ref[...])
    tot = msgp_ref[0] + msgp_ref[1] + hs_ref[...]
    h = jnp.maximum(tot * dis[:, None] + b_ref[...], 0.0)
    out_ref[...] = jnp.dot(
        h, w2_ref[...], preferred_element_type=jnp.float32) * dis[:, None]


def _tcc_body(msgp_ref, hs_ref, degp_ref, b_ref, wfct_ref, bfc_ref, out_ref):
    dis = _dis_from_degp(degp_ref[...])
    tot = msgp_ref[0] + msgp_ref[1] + hs_ref[...]
    h = jnp.maximum(tot * dis[:, None] + b_ref[...], 0.0)
    rows = lax.broadcasted_iota(jnp.int32, (NP, 1), 0)
    h = jnp.where(rows < NNODES, h, 0.0)
    g = jnp.sum(h, axis=0, keepdims=True) * (1.0 / NNODES)
    z = jnp.sum(g * wfct_ref[...], axis=1, keepdims=True) + bfc_ref[...]
    out_ref[...] = jax.nn.sigmoid(z)


def kernel(x, edge_index, W1, b1, W2, b2, Wfc, bfc):
    nedges = edge_index.shape[1]
    src = edge_index[0]
    dst = edge_index[1]
    x_pad = jnp.concatenate(
        [x, jnp.zeros((NP - NNODES, DIN), jnp.float32)], axis=0)

    ep = nedges // NW
    nfw = ep // CHUNK
    tail = ep - nfw * CHUNK
    fast = (nedges == ep * NW and ep % 8 == 0 and tail % 8 == 0
            and nfw >= 2 and nfw % 2 == 0)

    if fast:
        degp = _make_deg_fast(nfw, tail)(dst)
        msg_kernel = _make_msg_fast(nfw, tail)
        msg1_args = msg2_args = (src, dst)
    else:
        nwin = -(-nedges // (NW * CHUNK))
        epad = NW * nwin * CHUNK - nedges
        srcw = jnp.concatenate(
            [src, jnp.full((epad,), NNODES, jnp.int32)]
        ).reshape(NW, nwin, CHUNK)
        dstw = jnp.concatenate(
            [dst, jnp.full((epad,), NNODES + 8, jnp.int32)]
        ).reshape(NW, nwin, CHUNK)
        degp = _make_deg_kernel(nwin)(dstw)
        msg_kernel = _make_msg_kernel(nwin)
        msg1_args = msg2_args = (srcw, dstw)

    h1 = pl.pallas_call(
        _tca1_body,
        out_shape=jax.ShapeDtypeStruct((NP, DH), jnp.float32),
    )(x_pad, W1)

    hs1 = pl.pallas_call(
        _tca2_body,
        out_shape=jax.ShapeDtypeStruct((NP, DH), jnp.float32),
    )(h1, degp)

    msgp1 = msg_kernel(hs1, *msg1_args)

    hs2 = pl.pallas_call(
        _tcb_body,
        out_shape=jax.ShapeDtypeStruct((NP, DH), jnp.float32),
    )(msgp1, hs1, degp, b1.reshape(1, DH), W2)

    msgp2 = msg_kernel(hs2, *msg2_args)

    out = pl.pallas_call(
        _tcc_body,
        out_shape=jax.ShapeDtypeStruct((1, 1), jnp.float32),
    )(msgp2, hs2, degp, b2.reshape(1, DH), Wfc.T.reshape(1, DH),
      bfc.reshape(1, 1))

    return out.reshape(1)


# speedup vs baseline: 1.0184x; 1.0051x over previous
import functools

import jax
import jax.numpy as jnp
from jax import lax
from jax.experimental import pallas as pl
from jax.experimental.pallas import tpu as pltpu
from jax.experimental.pallas import tpu_sc as plsc

NNODES = 10000
DIN = 128
DH = 64
NC = 2
NS = 16
NW = NC * NS
CHUNK = 128
RPT = 632
NP = NS * RPT
DEGW = 16

def _mesh():
    return plsc.VectorSubcoreMesh(
        core_axis_name="c", subcore_axis_name="s",
        num_cores=NC, num_subcores=NS)


def _zero_rows(ref, nrows, width):
    zero16 = jnp.zeros((16,), jnp.float32)

    def zrow(i, _):
        for j in range(width // 16):
            ref[i, pl.ds(j * 16, 16)] = zero16
        return 0

    lax.fori_loop(0, nrows, zrow, 0)


def _zero_acc_slice(zbuf, acc_sh, sid):
    nfull = RPT // CHUNK
    rem = RPT - nfull * CHUNK
    for j in range(nfull):
        pltpu.sync_copy(zbuf, acc_sh.at[pl.ds(sid * RPT + j * CHUNK, CHUNK)])
    if rem:
        pltpu.sync_copy(
            zbuf.at[pl.ds(0, rem)],
            acc_sh.at[pl.ds(sid * RPT + nfull * CHUNK, rem)])


def _deg_fast_body(nfw, tail, dst_hbm, out_hbm, *refs):
    if tail:
        (di0, di1, ones_v, dit, ones_t, buf_v, acc_sh, isem0, isem1) = refs
    else:
        (di0, di1, ones_v, buf_v, acc_sh, isem0, isem1) = refs
    ep = nfw * CHUNK + tail
    npairs = nfw // 2
    cid = lax.axis_index("c")
    sid = lax.axis_index("s")
    wid = cid * NS + sid
    base = wid * ep
    one16 = jnp.ones((16,), jnp.float32)
    zero16 = jnp.zeros((16,), jnp.float32)

    pltpu.async_copy(dst_hbm.at[pl.ds(base, CHUNK)], di0, isem0)
    pltpu.async_copy(dst_hbm.at[pl.ds(base + CHUNK, CHUNK)], di1, isem1)

    def fill(i, _):
        ones_v[i] = one16
        buf_v[i] = zero16
        return 0

    lax.fori_loop(0, CHUNK, fill, 0)
    if tail:
        def fillt(i, _):
            ones_t[i] = one16
            return 0

        lax.fori_loop(0, tail, fillt, 0)
    _zero_acc_slice(buf_v, acc_sh, sid)
    pltpu.make_async_copy(dst_hbm.at[pl.ds(base, CHUNK)], di0, isem0).wait()
    plsc.subcore_barrier()

    def pair(p, _):
        n0 = base + (2 * p + 2) * CHUNK
        n1 = n0 + CHUNK
        pltpu.sync_copy(ones_v, acc_sh.at[di0], add=True)
        pltpu.async_copy(dst_hbm.at[pl.ds(n0, CHUNK)], di0, isem0)
        pltpu.make_async_copy(
            dst_hbm.at[pl.ds(n0, CHUNK)], di1, isem1).wait()
        pltpu.sync_copy(ones_v, acc_sh.at[di1], add=True)
        pltpu.make_async_copy(
            dst_hbm.at[pl.ds(n0, CHUNK)], di0, isem0).wait()
        pltpu.async_copy(dst_hbm.at[pl.ds(n1, CHUNK)], di1, isem1)
        return 0

    lax.fori_loop(0, npairs - 1, pair, 0)
    pltpu.sync_copy(ones_v, acc_sh.at[di0], add=True)
    if tail:
        pltpu.async_copy(
            dst_hbm.at[pl.ds(base + nfw * CHUNK, tail)], dit, isem0)
    pltpu.make_async_copy(
        dst_hbm.at[pl.ds(base, CHUNK)], di1, isem1).wait()
    pltpu.sync_copy(ones_v, acc_sh.at[di1], add=True)
    if tail:
        pltpu.make_async_copy(
            dst_hbm.at[pl.ds(base, tail)], dit, isem0).wait()
        pltpu.sync_copy(ones_t, acc_sh.at[dit], add=True)
    plsc.subcore_barrier()
    pltpu.sync_copy(acc_sh.at[pl.ds(sid * RPT, RPT)],
                    out_hbm.at[cid, pl.ds(sid * RPT, RPT)])


def _msg_fast_body(nfw, tail, hs_hbm, src_hbm, dst_hbm, out_hbm, *refs):
    if tail:
        (si0, di0, si1, di1, rows0, rows1, sit, dit, rowst,
         rd_v, acc_sh, gsem0, gsem1, isem0, isem1) = refs
    else:
        (si0, di0, si1, di1, rows0, rows1,
         rd_v, acc_sh, gsem0, gsem1, isem0, isem1) = refs
    ep = nfw * CHUNK + tail
    npairs = nfw // 2
    cid = lax.axis_index("c")
    sid = lax.axis_index("s")
    wid = cid * NS + sid
    base = wid * ep

    pltpu.async_copy(src_hbm.at[pl.ds(base, CHUNK)], si0, isem0)
    pltpu.async_copy(dst_hbm.at[pl.ds(base, CHUNK)], di0, isem0)
    pltpu.async_copy(src_hbm.at[pl.ds(base + CHUNK, CHUNK)], si1, isem1)
    pltpu.async_copy(dst_hbm.at[pl.ds(base + CHUNK, CHUNK)], di1, isem1)
    _zero_rows(rd_v, CHUNK, DH)
    pltpu.make_async_copy(src_hbm.at[pl.ds(base, CHUNK)], si0, isem0).wait()
    pltpu.make_async_copy(dst_hbm.at[pl.ds(base, CHUNK)], di0, isem0).wait()
    pltpu.async_copy(hs_hbm.at[si0], rows0, gsem0)
    _zero_acc_slice(rd_v, acc_sh, sid)
    plsc.subcore_barrier()

    def pair(p, _):
        b0 = base + (2 * p + 1) * CHUNK
        n0 = b0 + CHUNK
        n1 = n0 + CHUNK
        pltpu.make_async_copy(
            src_hbm.at[pl.ds(b0, CHUNK)], si1, isem1).wait()
        pltpu.make_async_copy(
            dst_hbm.at[pl.ds(b0, CHUNK)], di1, isem1).wait()
        pltpu.async_copy(hs_hbm.at[si1], rows1, gsem1)
        pltpu.make_async_copy(hs_hbm.at[si0], rows0, gsem0).wait()
        pltpu.sync_copy(rows0, acc_sh.at[di0], add=True)
        pltpu.async_copy(src_hbm.at[pl.ds(n0, CHUNK)], si0, isem0)
        pltpu.async_copy(dst_hbm.at[pl.ds(n0, CHUNK)], di0, isem0)
        pltpu.make_async_copy(hs_hbm.at[si1], rows1, gsem1).wait()
        pltpu.sync_copy(rows1, acc_sh.at[di1], add=True)
        pltpu.make_async_copy(
            src_hbm.at[pl.ds(n0, CHUNK)], si0, isem0).wait()
        pltpu.make_async_copy(
            dst_hbm.at[pl.ds(n0, CHUNK)], di0, isem0).wait()
        pltpu.async_copy(hs_hbm.at[si0], rows0, gsem0)
        pltpu.async_copy(src_hbm.at[pl.ds(n1, CHUNK)], si1, isem1)
        pltpu.async_copy(dst_hbm.at[pl.ds(n1, CHUNK)], di1, isem1)
        return 0

    lax.fori_loop(0, npairs - 1, pair, 0)

    bL = base + (nfw - 1) * CHUNK
    pltpu.make_async_copy(src_hbm.at[pl.ds(bL, CHUNK)], si1, isem1).wait()
    pltpu.make_async_copy(dst_hbm.at[pl.ds(bL, CHUNK)], di1, isem1).wait()
    pltpu.async_copy(hs_hbm.at[si1], rows1, gsem1)
    pltpu.make_async_copy(hs_hbm.at[si0], rows0, gsem0).wait()
    pltpu.sync_copy(rows0, acc_sh.at[di0], add=True)
    if tail:
        pltpu.async_copy(
            src_hbm.at[pl.ds(base + nfw * CHUNK, tail)], sit, isem0)
        pltpu.async_copy(
            dst_hbm.at[pl.ds(base + nfw * CHUNK, tail)], dit, isem0)
    pltpu.make_async_copy(hs_hbm.at[si1], rows1, gsem1).wait()
    pltpu.sync_copy(rows1, acc_sh.at[di1], add=True)
    if tail:
        pltpu.make_async_copy(
            src_hbm.at[pl.ds(base, tail)], sit, isem0).wait()
        pltpu.make_async_copy(
            dst_hbm.at[pl.ds(base, tail)], dit, isem0).wait()
        pltpu.async_copy(hs_hbm.at[sit], rowst, gsem0).wait()
        pltpu.sync_copy(rowst, acc_sh.at[dit], add=True)
    plsc.subcore_barrier()
    pltpu.sync_copy(acc_sh.at[pl.ds(sid * RPT, RPT)],
                    out_hbm.at[cid, pl.ds(sid * RPT, RPT)])


def _make_deg_fast(nfw, tail):
    scratch = [
        pltpu.VMEM((CHUNK,), jnp.int32),
        pltpu.VMEM((CHUNK,), jnp.int32),
        pltpu.VMEM((CHUNK, DEGW), jnp.float32),
    ]
    if tail:
        scratch += [
            pltpu.VMEM((tail,), jnp.int32),
            pltpu.VMEM((tail, DEGW), jnp.float32),
        ]
    scratch += [
        pltpu.VMEM((CHUNK, DEGW), jnp.float32),
        pltpu.VMEM_SHARED((NP, DEGW), jnp.float32),
        pltpu.SemaphoreType.DMA,
        pltpu.SemaphoreType.DMA,
    ]
    return pl.kernel(
        functools.partial(_deg_fast_body, nfw, tail),
        out_type=jax.ShapeDtypeStruct((NC, NP, DEGW), jnp.float32),
        mesh=_mesh(),
        scratch_types=scratch,
        compiler_params=pltpu.CompilerParams(use_tc_tiling_on_sc=False),
        name="gcn_degree_sc",
    )


def _make_msg_fast(nfw, tail):
    scratch = [
        pltpu.VMEM((CHUNK,), jnp.int32),
        pltpu.VMEM((CHUNK,), jnp.int32),
        pltpu.VMEM((CHUNK,), jnp.int32),
        pltpu.VMEM((CHUNK,), jnp.int32),
        pltpu.VMEM((CHUNK, DH), jnp.float32),
        pltpu.VMEM((CHUNK, DH), jnp.float32),
    ]
    if tail:
        scratch += [
            pltpu.VMEM((tail,), jnp.int32),
            pltpu.VMEM((tail,), jnp.int32),
            pltpu.VMEM((tail, DH), jnp.float32),
        ]
    scratch += [
        pltpu.VMEM((CHUNK, DH), jnp.float32),
        pltpu.VMEM_SHARED((NP, DH), jnp.float32),
        pltpu.SemaphoreType.DMA,
        pltpu.SemaphoreType.DMA,
        pltpu.SemaphoreType.DMA,
        pltpu.SemaphoreType.DMA,
    ]
    return pl.kernel(
        functools.partial(_msg_fast_body, nfw, tail),
        out_type=jax.ShapeDtypeStruct((NC, NP, DH), jnp.float32),
        mesh=_mesh(),
        scratch_types=scratch,
        compiler_params=pltpu.CompilerParams(use_tc_tiling_on_sc=False),
        name="gcn_message_sc",
    )


def _deg_body(nwin, dstw_hbm, out_hbm, di_v, ones_v, buf_v, acc_sh):
    cid = lax.axis_index("c")
    sid = lax.axis_index("s")
    wid = cid * NS + sid
    one16 = jnp.ones((16,), jnp.float32)

    def fill(i, _):
        ones_v[i] = one16
        return 0

    lax.fori_loop(0, CHUNK, fill, 0)
    _zero_rows(buf_v, RPT, DEGW)
    pltpu.sync_copy(buf_v, acc_sh.at[pl.ds(sid * RPT, RPT)])
    plsc.subcore_barrier()

    def step(k, _):
        pltpu.sync_copy(dstw_hbm.at[wid, k], di_v)
        pltpu.sync_copy(ones_v, acc_sh.at[di_v], add=True)
        return 0

    lax.fori_loop(0, nwin, step, 0)
    plsc.subcore_barrier()
    pltpu.sync_copy(acc_sh.at[pl.ds(sid * RPT, RPT)], buf_v)
    pltpu.sync_copy(buf_v, out_hbm.at[cid, pl.ds(sid * RPT, RPT)])


def _msg_body(nwin, hs_hbm, srcw_hbm, dstw_hbm, out_hbm,
              si_v, di_v, rows_v, rd_v, acc_sh, sem):
    cid = lax.axis_index("c")
    sid = lax.axis_index("s")
    wid = cid * NS + sid

    _zero_rows(rd_v, RPT, DH)
    pltpu.sync_copy(rd_v, acc_sh.at[pl.ds(sid * RPT, RPT)])
    plsc.subcore_barrier()

    def step(k, _):
        pltpu.sync_copy(srcw_hbm.at[wid, k], si_v)
        pltpu.sync_copy(dstw_hbm.at[wid, k], di_v)
        pltpu.async_copy(hs_hbm.at[si_v], rows_v, sem).wait()
        pltpu.sync_copy(rows_v, acc_sh.at[di_v], add=True)
        return 0

    lax.fori_loop(0, nwin, step, 0)
    plsc.subcore_barrier()
    pltpu.sync_copy(acc_sh.at[pl.ds(sid * RPT, RPT)], rd_v)
    pltpu.sync_copy(rd_v, out_hbm.at[cid, pl.ds(sid * RPT, RPT)])


def _make_deg_kernel(nwin):
    return pl.kernel(
        functools.partial(_deg_body, nwin),
        out_type=jax.ShapeDtypeStruct((NC, NP, DEGW), jnp.float32),
        mesh=_mesh(),
        scratch_types=[
            pltpu.VMEM((CHUNK,), jnp.int32),
            pltpu.VMEM((CHUNK, DEGW), jnp.float32),
            pltpu.VMEM((RPT, DEGW), jnp.float32),
            pltpu.VMEM_SHARED((NP, DEGW), jnp.float32),
        ],
        compiler_params=pltpu.CompilerParams(use_tc_tiling_on_sc=False),
        name="gcn_degree_sc",
    )


def _make_msg_kernel(nwin):
    return pl.kernel(
        functools.partial(_msg_body, nwin),
        out_type=jax.ShapeDtypeStruct((NC, NP, DH), jnp.float32),
        mesh=_mesh(),
        scratch_types=[
            pltpu.VMEM((CHUNK,), jnp.int32),
            pltpu.VMEM((CHUNK,), jnp.int32),
            pltpu.VMEM((CHUNK, DH), jnp.float32),
            pltpu.VMEM((RPT, DH), jnp.float32),
            pltpu.VMEM_SHARED((NP, DH), jnp.float32),
            pltpu.SemaphoreType.DMA,
        ],
        compiler_params=pltpu.CompilerParams(use_tc_tiling_on_sc=False),
        name="gcn_message_sc",
    )


def _dis_from_degp(degp):
    return lax.rsqrt(degp[0, :, 0] + degp[1, :, 0] + 1.0)


def _tca_body(x_ref, w1_ref, degp_ref, hs_ref):
    dis = _dis_from_degp(degp_ref[...])
    h = jnp.dot(x_ref[...], w1_ref[...], preferred_element_type=jnp.float32)
    hs_ref[...] = h * dis[:, None]


def _tcb_body(msgp_ref, hs_ref, degp_ref, b_ref, w2_ref, out_ref):
    dis = _dis_from_degp(degp_ref[...])
    tot = msgp_ref[0] + msgp_ref[1] + hs_ref[...]
    h = jnp.maximum(tot * dis[:, None] + b_ref[...], 0.0)
    out_ref[...] = jnp.dot(
        h, w2_ref[...], preferred_element_type=jnp.float32) * dis[:, None]


def _tcc_body(msgp_ref, hs_ref, degp_ref, b_ref, wfct_ref, bfc_ref, out_ref):
    dis = _dis_from_degp(degp_ref[...])
    tot = msgp_ref[0] + msgp_ref[1] + hs_ref[...]
    h = jnp.maximum(tot * dis[:, None] + b_ref[...], 0.0)
    rows = lax.broadcasted_iota(jnp.int32, (NP, 1), 0)
    h = jnp.where(rows < NNODES, h, 0.0)
    g = jnp.sum(h, axis=0, keepdims=True) * (1.0 / NNODES)
    z = jnp.sum(g * wfct_ref[...], axis=1, keepdims=True) + bfc_ref[...]
    out_ref[...] = jax.nn.sigmoid(z)


def kernel(x, edge_index, W1, b1, W2, b2, Wfc, bfc):
    nedges = edge_index.shape[1]
    src = edge_index[0]
    dst = edge_index[1]
    x_pad = jnp.concatenate(
        [x, jnp.zeros((NP - NNODES, DIN), jnp.float32)], axis=0)

    ep = nedges // NW
    nfw = ep // CHUNK
    tail = ep - nfw * CHUNK
    fast = (nedges == ep * NW and ep % 8 == 0 and tail % 8 == 0
            and nfw >= 2 and nfw % 2 == 0)

    if fast:
        degp = _make_deg_fast(nfw, tail)(dst)
        msg_kernel = _make_msg_fast(nfw, tail)
        msg1_args = msg2_args = (src, dst)
    else:
        nwin = -(-nedges // (NW * CHUNK))
        epad = NW * nwin * CHUNK - nedges
        srcw = jnp.concatenate(
            [src, jnp.full((epad,), NNODES, jnp.int32)]
        ).reshape(NW, nwin, CHUNK)
        dstw = jnp.concatenate(
            [dst, jnp.full((epad,), NNODES + 8, jnp.int32)]
        ).reshape(NW, nwin, CHUNK)
        degp = _make_deg_kernel(nwin)(dstw)
        msg_kernel = _make_msg_kernel(nwin)
        msg1_args = msg2_args = (srcw, dstw)

    hs1 = pl.pallas_call(
        _tca_body,
        out_shape=jax.ShapeDtypeStruct((NP, DH), jnp.float32),
    )(x_pad, W1, degp)

    msgp1 = msg_kernel(hs1, *msg1_args)

    hs2 = pl.pallas_call(
        _tcb_body,
        out_shape=jax.ShapeDtypeStruct((NP, DH), jnp.float32),
    )(msgp1, hs1, degp, b1.reshape(1, DH), W2)

    msgp2 = msg_kernel(hs2, *msg2_args)

    out = pl.pallas_call(
        _tcc_body,
        out_shape=jax.ShapeDtypeStruct((1, 1), jnp.float32),
    )(msgp2, hs2, degp, b2.reshape(1, DH), Wfc.T.reshape(1, DH),
      bfc.reshape(1, 1))

    return out.reshape(1)


# speedup vs baseline: 1.0241x; 1.0056x over previous
import functools

import jax
import jax.numpy as jnp
from jax import lax
from jax.experimental import pallas as pl
from jax.experimental.pallas import tpu as pltpu
from jax.experimental.pallas import tpu_sc as plsc

NNODES = 10000
DIN = 128
DH = 64
NC = 2
NS = 16
NW = NC * NS
CHUNK = 128
RPT = 632
NP = NS * RPT
DEGW = 16

def _mesh():
    return plsc.VectorSubcoreMesh(
        core_axis_name="c", subcore_axis_name="s",
        num_cores=NC, num_subcores=NS)


def _zero_rows(ref, nrows, width):
    zero16 = jnp.zeros((16,), jnp.float32)

    def zrow(i, _):
        for j in range(width // 16):
            ref[i, pl.ds(j * 16, 16)] = zero16
        return 0

    lax.fori_loop(0, nrows, zrow, 0)


def _zero_acc_slice(zbuf, acc_sh, sid):
    nfull = RPT // CHUNK
    rem = RPT - nfull * CHUNK
    for j in range(nfull):
        pltpu.sync_copy(zbuf, acc_sh.at[pl.ds(sid * RPT + j * CHUNK, CHUNK)])
    if rem:
        pltpu.sync_copy(
            zbuf.at[pl.ds(0, rem)],
            acc_sh.at[pl.ds(sid * RPT + nfull * CHUNK, rem)])


def _deg_fast_body(nfw, tail, dst_hbm, out_hbm, *refs):
    if tail:
        (di0, di1, ones_v, dit, ones_t, buf_v, acc_sh, isem0, isem1) = refs
    else:
        (di0, di1, ones_v, buf_v, acc_sh, isem0, isem1) = refs
    ep = nfw * CHUNK + tail
    npairs = nfw // 2
    cid = lax.axis_index("c")
    sid = lax.axis_index("s")
    wid = cid * NS + sid
    base = wid * ep
    one16 = jnp.ones((16,), jnp.float32)
    zero16 = jnp.zeros((16,), jnp.float32)

    pltpu.async_copy(dst_hbm.at[pl.ds(base, CHUNK)], di0, isem0)
    pltpu.async_copy(dst_hbm.at[pl.ds(base + CHUNK, CHUNK)], di1, isem1)

    def fill(i, _):
        ones_v[i] = one16
        buf_v[i] = zero16
        return 0

    lax.fori_loop(0, CHUNK, fill, 0)
    if tail:
        def fillt(i, _):
            ones_t[i] = one16
            return 0

        lax.fori_loop(0, tail, fillt, 0)
    _zero_acc_slice(buf_v, acc_sh, sid)
    pltpu.make_async_copy(dst_hbm.at[pl.ds(base, CHUNK)], di0, isem0).wait()
    plsc.subcore_barrier()

    def pair(p, _):
        n0 = base + (2 * p + 2) * CHUNK
        n1 = n0 + CHUNK
        pltpu.sync_copy(ones_v, acc_sh.at[di0], add=True)
        pltpu.async_copy(dst_hbm.at[pl.ds(n0, CHUNK)], di0, isem0)
        pltpu.make_async_copy(
            dst_hbm.at[pl.ds(n0, CHUNK)], di1, isem1).wait()
        pltpu.sync_copy(ones_v, acc_sh.at[di1], add=True)
        pltpu.make_async_copy(
            dst_hbm.at[pl.ds(n0, CHUNK)], di0, isem0).wait()
        pltpu.async_copy(dst_hbm.at[pl.ds(n1, CHUNK)], di1, isem1)
        return 0

    lax.fori_loop(0, npairs - 1, pair, 0)
    pltpu.sync_copy(ones_v, acc_sh.at[di0], add=True)
    if tail:
        pltpu.async_copy(
            dst_hbm.at[pl.ds(base + nfw * CHUNK, tail)], dit, isem0)
    pltpu.make_async_copy(
        dst_hbm.at[pl.ds(base, CHUNK)], di1, isem1).wait()
    pltpu.sync_copy(ones_v, acc_sh.at[di1], add=True)
    if tail:
        pltpu.make_async_copy(
            dst_hbm.at[pl.ds(base, tail)], dit, isem0).wait()
        pltpu.sync_copy(ones_t, acc_sh.at[dit], add=True)
    plsc.subcore_barrier()
    pltpu.sync_copy(acc_sh.at[pl.ds(sid * RPT, RPT)],
                    out_hbm.at[cid, pl.ds(sid * RPT, RPT)])


def _msg_fast_body(nfw, tail, hs_hbm, src_hbm, dst_hbm, out_hbm, *refs):
    if tail:
        (si0, di0, si1, di1, rows0, rows1, sit, dit, rowst,
         rd_v, acc_sh, gsem0, gsem1, isem0, isem1) = refs
    else:
        (si0, di0, si1, di1, rows0, rows1,
         rd_v, acc_sh, gsem0, gsem1, isem0, isem1) = refs
    ep = nfw * CHUNK + tail
    npairs = nfw // 2
    cid = lax.axis_index("c")
    sid = lax.axis_index("s")
    wid = cid * NS + sid
    base = wid * ep

    pltpu.async_copy(src_hbm.at[pl.ds(base, CHUNK)], si0, isem0)
    pltpu.async_copy(dst_hbm.at[pl.ds(base, CHUNK)], di0, isem0)
    pltpu.async_copy(src_hbm.at[pl.ds(base + CHUNK, CHUNK)], si1, isem1)
    pltpu.async_copy(dst_hbm.at[pl.ds(base + CHUNK, CHUNK)], di1, isem1)
    _zero_rows(rd_v, CHUNK, DH)
    pltpu.make_async_copy(src_hbm.at[pl.ds(base, CHUNK)], si0, isem0).wait()
    pltpu.make_async_copy(dst_hbm.at[pl.ds(base, CHUNK)], di0, isem0).wait()
    pltpu.async_copy(hs_hbm.at[si0], rows0, gsem0)
    _zero_acc_slice(rd_v, acc_sh, sid)
    plsc.subcore_barrier()

    def pair(p, _):
        b0 = base + (2 * p + 1) * CHUNK
        n0 = b0 + CHUNK
        n1 = n0 + CHUNK
        pltpu.make_async_copy(
            src_hbm.at[pl.ds(b0, CHUNK)], si1, isem1).wait()
        pltpu.make_async_copy(
            dst_hbm.at[pl.ds(b0, CHUNK)], di1, isem1).wait()
        pltpu.async_copy(hs_hbm.at[si1], rows1, gsem1)
        pltpu.make_async_copy(hs_hbm.at[si0], rows0, gsem0).wait()
        pltpu.sync_copy(rows0, acc_sh.at[di0], add=True)
        pltpu.async_copy(src_hbm.at[pl.ds(n0, CHUNK)], si0, isem0)
        pltpu.async_copy(dst_hbm.at[pl.ds(n0, CHUNK)], di0, isem0)
        pltpu.make_async_copy(hs_hbm.at[si1], rows1, gsem1).wait()
        pltpu.sync_copy(rows1, acc_sh.at[di1], add=True)
        pltpu.make_async_copy(
            src_hbm.at[pl.ds(n0, CHUNK)], si0, isem0).wait()
        pltpu.make_async_copy(
            dst_hbm.at[pl.ds(n0, CHUNK)], di0, isem0).wait()
        pltpu.async_copy(hs_hbm.at[si0], rows0, gsem0)
        pltpu.async_copy(src_hbm.at[pl.ds(n1, CHUNK)], si1, isem1)
        pltpu.async_copy(dst_hbm.at[pl.ds(n1, CHUNK)], di1, isem1)
        return 0

    lax.fori_loop(0, npairs - 1, pair, 0)

    bL = base + (nfw - 1) * CHUNK
    pltpu.make_async_copy(src_hbm.at[pl.ds(bL, CHUNK)], si1, isem1).wait()
    pltpu.make_async_copy(dst_hbm.at[pl.ds(bL, CHUNK)], di1, isem1).wait()
    pltpu.async_copy(hs_hbm.at[si1], rows1, gsem1)
    pltpu.make_async_copy(hs_hbm.at[si0], rows0, gsem0).wait()
    pltpu.sync_copy(rows0, acc_sh.at[di0], add=True)
    if tail:
        pltpu.async_copy(
            src_hbm.at[pl.ds(base + nfw * CHUNK, tail)], sit, isem0)
        pltpu.async_copy(
            dst_hbm.at[pl.ds(base + nfw * CHUNK, tail)], dit, isem0)
    pltpu.make_async_copy(hs_hbm.at[si1], rows1, gsem1).wait()
    pltpu.sync_copy(rows1, acc_sh.at[di1], add=True)
    if tail:
        pltpu.make_async_copy(
            src_hbm.at[pl.ds(base, tail)], sit, isem0).wait()
        pltpu.make_async_copy(
            dst_hbm.at[pl.ds(base, tail)], dit, isem0).wait()
        pltpu.async_copy(hs_hbm.at[sit], rowst, gsem0).wait()
        pltpu.sync_copy(rowst, acc_sh.at[dit], add=True)
    plsc.subcore_barrier()
    pltpu.sync_copy(acc_sh.at[pl.ds(sid * RPT, RPT)],
                    out_hbm.at[cid, pl.ds(sid * RPT, RPT)])


def _make_deg_fast(nfw, tail):
    scratch = [
        pltpu.VMEM((CHUNK,), jnp.int32),
        pltpu.VMEM((CHUNK,), jnp.int32),
        pltpu.VMEM((CHUNK, DEGW), jnp.float32),
    ]
    if tail:
        scratch += [
            pltpu.VMEM((tail,), jnp.int32),
            pltpu.VMEM((tail, DEGW), jnp.float32),
        ]
    scratch += [
        pltpu.VMEM((CHUNK, DEGW), jnp.float32),
        pltpu.VMEM_SHARED((NP, DEGW), jnp.float32),
        pltpu.SemaphoreType.DMA,
        pltpu.SemaphoreType.DMA,
    ]
    return pl.kernel(
        functools.partial(_deg_fast_body, nfw, tail),
        out_type=jax.ShapeDtypeStruct((NC, NP, DEGW), jnp.float32),
        mesh=_mesh(),
        scratch_types=scratch,
        compiler_params=pltpu.CompilerParams(use_tc_tiling_on_sc=False),
        name="gcn_degree_sc",
    )


def _make_msg_fast(nfw, tail):
    scratch = [
        pltpu.VMEM((CHUNK,), jnp.int32),
        pltpu.VMEM((CHUNK,), jnp.int32),
        pltpu.VMEM((CHUNK,), jnp.int32),
        pltpu.VMEM((CHUNK,), jnp.int32),
        pltpu.VMEM((CHUNK, DH), jnp.float32),
        pltpu.VMEM((CHUNK, DH), jnp.float32),
    ]
    if tail:
        scratch += [
            pltpu.VMEM((tail,), jnp.int32),
            pltpu.VMEM((tail,), jnp.int32),
            pltpu.VMEM((tail, DH), jnp.float32),
        ]
    scratch += [
        pltpu.VMEM((CHUNK, DH), jnp.float32),
        pltpu.VMEM_SHARED((NP, DH), jnp.float32),
        pltpu.SemaphoreType.DMA,
        pltpu.SemaphoreType.DMA,
        pltpu.SemaphoreType.DMA,
        pltpu.SemaphoreType.DMA,
    ]
    return pl.kernel(
        functools.partial(_msg_fast_body, nfw, tail),
        out_type=jax.ShapeDtypeStruct((NC, NP, DH), jnp.float32),
        mesh=_mesh(),
        scratch_types=scratch,
        compiler_params=pltpu.CompilerParams(use_tc_tiling_on_sc=False),
        name="gcn_message_sc",
    )


def _deg_body(nwin, dstw_hbm, out_hbm, di_v, ones_v, buf_v, acc_sh):
    cid = lax.axis_index("c")
    sid = lax.axis_index("s")
    wid = cid * NS + sid
    one16 = jnp.ones((16,), jnp.float32)

    def fill(i, _):
        ones_v[i] = one16
        return 0

    lax.fori_loop(0, CHUNK, fill, 0)
    _zero_rows(buf_v, RPT, DEGW)
    pltpu.sync_copy(buf_v, acc_sh.at[pl.ds(sid * RPT, RPT)])
    plsc.subcore_barrier()

    def step(k, _):
        pltpu.sync_copy(dstw_hbm.at[wid, k], di_v)
        pltpu.sync_copy(ones_v, acc_sh.at[di_v], add=True)
        return 0

    lax.fori_loop(0, nwin, step, 0)
    plsc.subcore_barrier()
    pltpu.sync_copy(acc_sh.at[pl.ds(sid * RPT, RPT)], buf_v)
    pltpu.sync_copy(buf_v, out_hbm.at[cid, pl.ds(sid * RPT, RPT)])


def _msg_body(nwin, hs_hbm, srcw_hbm, dstw_hbm, out_hbm,
              si_v, di_v, rows_v, rd_v, acc_sh, sem):
    cid = lax.axis_index("c")
    sid = lax.axis_index("s")
    wid = cid * NS + sid

    _zero_rows(rd_v, RPT, DH)
    pltpu.sync_copy(rd_v, acc_sh.at[pl.ds(sid * RPT, RPT)])
    plsc.subcore_barrier()

    def step(k, _):
        pltpu.sync_copy(srcw_hbm.at[wid, k], si_v)
        pltpu.sync_copy(dstw_hbm.at[wid, k], di_v)
        pltpu.async_copy(hs_hbm.at[si_v], rows_v, sem).wait()
        pltpu.sync_copy(rows_v, acc_sh.at[di_v], add=True)
        return 0

    lax.fori_loop(0, nwin, step, 0)
    plsc.subcore_barrier()
    pltpu.sync_copy(acc_sh.at[pl.ds(sid * RPT, RPT)], rd_v)
    pltpu.sync_copy(rd_v, out_hbm.at[cid, pl.ds(sid * RPT, RPT)])


def _make_deg_kernel(nwin):
    return pl.kernel(
        functools.partial(_deg_body, nwin),
        out_type=jax.ShapeDtypeStruct((NC, NP, DEGW), jnp.float32),
        mesh=_mesh(),
        scratch_types=[
            pltpu.VMEM((CHUNK,), jnp.int32),
            pltpu.VMEM((CHUNK, DEGW), jnp.float32),
            pltpu.VMEM((RPT, DEGW), jnp.float32),
            pltpu.VMEM_SHARED((NP, DEGW), jnp.float32),
        ],
        compiler_params=pltpu.CompilerParams(use_tc_tiling_on_sc=False),
        name="gcn_degree_sc",
    )


def _make_msg_kernel(nwin):
    return pl.kernel(
        functools.partial(_msg_body, nwin),
        out_type=jax.ShapeDtypeStruct((NC, NP, DH), jnp.float32),
        mesh=_mesh(),
        scratch_types=[
            pltpu.VMEM((CHUNK,), jnp.int32),
            pltpu.VMEM((CHUNK,), jnp.int32),
            pltpu.VMEM((CHUNK, DH), jnp.float32),
            pltpu.VMEM((RPT, DH), jnp.float32),
            pltpu.VMEM_SHARED((NP, DH), jnp.float32),
            pltpu.SemaphoreType.DMA,
        ],
        compiler_params=pltpu.CompilerParams(use_tc_tiling_on_sc=False),
        name="gcn_message_sc",
    )


def _dis_from_degp(degp):
    return lax.rsqrt(degp[0, :, 0] + degp[1, :, 0] + 1.0)


def _tca_body(x_ref, w1_ref, degp_ref, hs_ref):
    dis = _dis_from_degp(degp_ref[...])
    h = jnp.dot(x_ref[...], w1_ref[...], preferred_element_type=jnp.float32)
    hs_ref[pl.ds(0, NNODES), :] = h * dis[:NNODES, None]
    hs_ref[pl.ds(NNODES, NP - NNODES), :] = jnp.zeros(
        (NP - NNODES, DH), jnp.float32)


def _tcb_body(msgp_ref, hs_ref, degp_ref, b_ref, w2_ref, out_ref):
    dis = _dis_from_degp(degp_ref[...])
    tot = msgp_ref[0] + msgp_ref[1] + hs_ref[...]
    h = jnp.maximum(tot * dis[:, None] + b_ref[...], 0.0)
    out_ref[...] = jnp.dot(
        h, w2_ref[...], preferred_element_type=jnp.float32) * dis[:, None]


def _tcc_body(msgp_ref, hs_ref, degp_ref, b_ref, wfct_ref, bfc_ref, out_ref):
    dis = _dis_from_degp(degp_ref[...])
    tot = msgp_ref[0] + msgp_ref[1] + hs_ref[...]
    h = jnp.maximum(tot * dis[:, None] + b_ref[...], 0.0)
    rows = lax.broadcasted_iota(jnp.int32, (NP, 1), 0)
    h = jnp.where(rows < NNODES, h, 0.0)
    g = jnp.sum(h, axis=0, keepdims=True) * (1.0 / NNODES)
    z = jnp.sum(g * wfct_ref[...], axis=1, keepdims=True) + bfc_ref[...]
    out_ref[...] = jax.nn.sigmoid(z)


def kernel(x, edge_index, W1, b1, W2, b2, Wfc, bfc):
    nedges = edge_index.shape[1]
    src = edge_index[0]
    dst = edge_index[1]

    ep = nedges // NW
    nfw = ep // CHUNK
    tail = ep - nfw * CHUNK
    fast = (nedges == ep * NW and ep % 8 == 0 and tail % 8 == 0
            and nfw >= 2 and nfw % 2 == 0)

    if fast:
        degp = _make_deg_fast(nfw, tail)(dst)
        msg_kernel = _make_msg_fast(nfw, tail)
        msg1_args = msg2_args = (src, dst)
    else:
        nwin = -(-nedges // (NW * CHUNK))
        epad = NW * nwin * CHUNK - nedges
        srcw = jnp.concatenate(
            [src, jnp.full((epad,), NNODES, jnp.int32)]
        ).reshape(NW, nwin, CHUNK)
        dstw = jnp.concatenate(
            [dst, jnp.full((epad,), NNODES + 8, jnp.int32)]
        ).reshape(NW, nwin, CHUNK)
        degp = _make_deg_kernel(nwin)(dstw)
        msg_kernel = _make_msg_kernel(nwin)
        msg1_args = msg2_args = (srcw, dstw)

    hs1 = pl.pallas_call(
        _tca_body,
        out_shape=jax.ShapeDtypeStruct((NP, DH), jnp.float32),
    )(x, W1, degp)

    msgp1 = msg_kernel(hs1, *msg1_args)

    hs2 = pl.pallas_call(
        _tcb_body,
        out_shape=jax.ShapeDtypeStruct((NP, DH), jnp.float32),
    )(msgp1, hs1, degp, b1.reshape(1, DH), W2)

    msgp2 = msg_kernel(hs2, *msg2_args)

    out = pl.pallas_call(
        _tcc_body,
        out_shape=jax.ShapeDtypeStruct((1, 1), jnp.float32),
    )(msgp2, hs2, degp, b2.reshape(1, DH), Wfc.T.reshape(1, DH),
      bfc.reshape(1, 1))

    return out.reshape(1)
